# baseline (speedup 1.0000x reference)
import functools

import jax
import jax.numpy as jnp
from jax import lax
from jax.experimental import pallas as pl
from jax.experimental.pallas import tpu as pltpu
from jax.experimental.pallas import tpu_sc as plsc

_THRESHOLD = 0.1
_NC = 2
_NS = 16
_L = 16
_CHUNK = 1024
_BSC = 32
_W = 2048


def _sc_body(x_hbm, t_hbm, out_hbm, xbuf0, xbuf1, tbuf0, tbuf1, cntbuf,
             sem0, sem1, *, num_classes, graph, n_batches):
    wid = lax.axis_index("s") * _NC + lax.axis_index("c")
    nw = _NC * _NS
    chunks_per_b = graph // _CHUNK
    n_chunks = n_batches * chunks_per_b // nw

    def start_chunk(c, xb, tb, sem):
        cg = wid + c * nw
        b = cg // chunks_per_b
        g0 = (cg % chunks_per_b) * _CHUNK
        pltpu.make_async_copy(x_hbm.at[b, :, pl.ds(g0, _CHUNK)], xb, sem).start()
        pltpu.make_async_copy(t_hbm.at[b, pl.ds(g0, _CHUNK)], tb, sem).start()

    def wait_chunk(xb, tb, sem):
        pltpu.make_async_copy(x_hbm.at[0, :, pl.ds(0, _CHUNK)], xb, sem).wait()
        pltpu.make_async_copy(t_hbm.at[0, pl.ds(0, _CHUNK)], tb, sem).wait()

    def compute(xb, tb, cnt):
        def lane_body(j, cnt):
            m1 = jnp.full((_L,), -jnp.inf, jnp.float32)
            m2 = jnp.full((_L,), -jnp.inf, jnp.float32)
            i1 = jnp.zeros((_L,), jnp.int32)
            i2 = jnp.zeros((_L,), jnp.int32)
            for k in range(num_classes):
                v = xb[k, pl.ds(j * _L, _L)]
                gt1 = v > m1
                gt2 = v > m2
                i2 = jnp.where(gt1, i1, jnp.where(gt2, k, i2))
                i1 = jnp.where(gt1, k, i1)
                m2 = jnp.maximum(m2, jnp.minimum(v, m1))
                m1 = jnp.maximum(v, m1)
            tv = tb[pl.ds(j * _L, _L)]
            one = jnp.ones((_L,), jnp.int32)
            zero = jnp.zeros((_L,), jnp.int32)
            c1 = jnp.where(i1 == tv, one, zero)
            sec = jnp.logical_and(m1 - m2 < _THRESHOLD, i2 == tv)
            c2 = jnp.where(sec, one, zero)
            return cnt + c1 + c2

        return lax.fori_loop(0, _CHUNK // _L, lane_body, cnt)

    start_chunk(0, xbuf0, tbuf0, sem0)
    start_chunk(1, xbuf1, tbuf1, sem1)

    def pair_body(p, cnt):
        c = 2 * p
        wait_chunk(xbuf0, tbuf0, sem0)
        cnt = compute(xbuf0, tbuf0, cnt)

        @pl.when(c + 2 < n_chunks)
        def _():
            start_chunk(c + 2, xbuf0, tbuf0, sem0)

        wait_chunk(xbuf1, tbuf1, sem1)
        cnt = compute(xbuf1, tbuf1, cnt)

        @pl.when(c + 3 < n_chunks)
        def _():
            start_chunk(c + 3, xbuf1, tbuf1, sem1)

        return cnt

    cnt = lax.fori_loop(0, n_chunks // 2, pair_body,
                        jnp.zeros((_L,), jnp.int32))
    cntbuf[...] = cnt
    pltpu.sync_copy(cntbuf, out_hbm.at[wid])


def _tc_kernel(x_ref, t_ref, o_ref, *, num_classes):
    first = jnp.logical_and(pl.program_id(0) == 0, pl.program_id(1) == 0)
    x = x_ref[0]
    tv = t_ref[0]
    ids = lax.broadcasted_iota(jnp.int32, x.shape, 0)
    m1 = jnp.max(x, axis=0, keepdims=True)
    i1 = jnp.min(jnp.where(x == m1, ids, num_classes), axis=0, keepdims=True)
    x2 = jnp.where(ids == i1, -jnp.inf, x)
    m2 = jnp.max(x2, axis=0, keepdims=True)
    i2 = jnp.min(jnp.where(x2 == m2, ids, num_classes), axis=0, keepdims=True)
    c1 = (i1 == tv).astype(jnp.int32)
    c2 = jnp.logical_and(m1 - m2 < _THRESHOLD, i2 == tv).astype(jnp.int32)
    cnt = jnp.sum(c1) + jnp.sum(c2)

    @pl.when(first)
    def _init():
        o_ref[0, 0] = 0

    o_ref[0, 0] += cnt


def kernel(input, target):
    batch, num_classes, graph = input.shape
    nw = _NC * _NS

    sc_body = functools.partial(
        _sc_body, num_classes=num_classes, graph=graph, n_batches=_BSC)
    sc_partials = pl.kernel(
        sc_body,
        out_type=jax.ShapeDtypeStruct((nw, _L), jnp.int32),
        scratch_types=[
            pltpu.VMEM((num_classes, _CHUNK), jnp.float32),
            pltpu.VMEM((num_classes, _CHUNK), jnp.float32),
            pltpu.VMEM((_CHUNK,), jnp.int32),
            pltpu.VMEM((_CHUNK,), jnp.int32),
            pltpu.VMEM((_L,), jnp.int32),
            pltpu.SemaphoreType.DMA,
            pltpu.SemaphoreType.DMA,
        ],
        mesh=plsc.VectorSubcoreMesh(core_axis_name="c", subcore_axis_name="s"),
    )(input, target)

    t3 = target.reshape(batch, 1, graph)
    n_tc = batch - _BSC
    nj = graph // _W
    tc_cnt = pl.pallas_call(
        functools.partial(_tc_kernel, num_classes=num_classes),
        grid=(n_tc, nj),
        in_specs=[
            pl.BlockSpec((1, num_classes, _W), lambda b, j: (_BSC + b, 0, j)),
            pl.BlockSpec((1, 1, _W), lambda b, j: (_BSC + b, 0, j)),
        ],
        out_specs=pl.BlockSpec(
            (1, 1), lambda b, j: (0, 0), memory_space=pltpu.SMEM
        ),
        out_shape=jax.ShapeDtypeStruct((1, 1), jnp.int32),
    )(input, t3)

    cnt = jnp.sum(sc_partials) + tc_cnt[0, 0]
    edge_acc = cnt.astype(jnp.float32) / float(target.size)
    return 1.0 - edge_acc


# speedup vs baseline: 1.1207x; 1.1207x over previous
import functools

import jax
import jax.numpy as jnp
from jax import lax
from jax.experimental import pallas as pl
from jax.experimental.pallas import tpu as pltpu
from jax.experimental.pallas import tpu_sc as plsc

_THRESHOLD = 0.1
_NC = 2
_NS = 16
_L = 16
_CHUNK = 1024
_BSC = 48
_W = 2048


def _sc_body(x_hbm, t_hbm, out_hbm, xbuf0, xbuf1, tbuf0, tbuf1, cntbuf,
             sem0, sem1, *, num_classes, graph, n_batches):
    wid = lax.axis_index("s") * _NC + lax.axis_index("c")
    nw = _NC * _NS
    chunks_per_b = graph // _CHUNK
    n_chunks = n_batches * chunks_per_b // nw

    def start_chunk(c, xb, tb, sem):
        cg = wid + c * nw
        b = cg // chunks_per_b
        g0 = (cg % chunks_per_b) * _CHUNK
        pltpu.make_async_copy(x_hbm.at[b, :, pl.ds(g0, _CHUNK)], xb, sem).start()
        pltpu.make_async_copy(t_hbm.at[b, pl.ds(g0, _CHUNK)], tb, sem).start()

    def wait_chunk(xb, tb, sem):
        pltpu.make_async_copy(x_hbm.at[0, :, pl.ds(0, _CHUNK)], xb, sem).wait()
        pltpu.make_async_copy(t_hbm.at[0, pl.ds(0, _CHUNK)], tb, sem).wait()

    def compute(xb, tb, cnt):
        def lane_body(j, cnt):
            m1 = jnp.full((_L,), -jnp.inf, jnp.float32)
            m2 = jnp.full((_L,), -jnp.inf, jnp.float32)
            i1 = jnp.zeros((_L,), jnp.int32)
            i2 = jnp.zeros((_L,), jnp.int32)
            for k in range(num_classes):
                v = xb[k, pl.ds(j * _L, _L)]
                gt1 = v > m1
                gt2 = v > m2
                i2 = jnp.where(gt1, i1, jnp.where(gt2, k, i2))
                i1 = jnp.where(gt1, k, i1)
                m2 = jnp.maximum(m2, jnp.minimum(v, m1))
                m1 = jnp.maximum(v, m1)
            tv = tb[pl.ds(j * _L, _L)]
            one = jnp.ones((_L,), jnp.int32)
            zero = jnp.zeros((_L,), jnp.int32)
            c1 = jnp.where(i1 == tv, one, zero)
            sec = jnp.logical_and(m1 - m2 < _THRESHOLD, i2 == tv)
            c2 = jnp.where(sec, one, zero)
            return cnt + c1 + c2

        return lax.fori_loop(0, _CHUNK // _L, lane_body, cnt)

    start_chunk(0, xbuf0, tbuf0, sem0)
    start_chunk(1, xbuf1, tbuf1, sem1)

    def pair_body(p, cnt):
        c = 2 * p
        wait_chunk(xbuf0, tbuf0, sem0)
        cnt = compute(xbuf0, tbuf0, cnt)

        @pl.when(c + 2 < n_chunks)
        def _():
            start_chunk(c + 2, xbuf0, tbuf0, sem0)

        wait_chunk(xbuf1, tbuf1, sem1)
        cnt = compute(xbuf1, tbuf1, cnt)

        @pl.when(c + 3 < n_chunks)
        def _():
            start_chunk(c + 3, xbuf1, tbuf1, sem1)

        return cnt

    cnt = lax.fori_loop(0, n_chunks // 2, pair_body,
                        jnp.zeros((_L,), jnp.int32))
    cntbuf[...] = cnt
    pltpu.sync_copy(cntbuf, out_hbm.at[wid])


def _tc_kernel(x_ref, t_ref, o_ref, *, num_classes):
    first = jnp.logical_and(pl.program_id(0) == 0, pl.program_id(1) == 0)
    x = x_ref[0]
    tv = t_ref[0]
    ids = lax.broadcasted_iota(jnp.int32, x.shape, 0)
    m1 = jnp.max(x, axis=0, keepdims=True)
    i1 = jnp.min(jnp.where(x == m1, ids, num_classes), axis=0, keepdims=True)
    x2 = jnp.where(ids == i1, -jnp.inf, x)
    m2 = jnp.max(x2, axis=0, keepdims=True)
    i2 = jnp.min(jnp.where(x2 == m2, ids, num_classes), axis=0, keepdims=True)
    c1 = (i1 == tv).astype(jnp.int32)
    c2 = jnp.logical_and(m1 - m2 < _THRESHOLD, i2 == tv).astype(jnp.int32)
    cnt = jnp.sum(c1) + jnp.sum(c2)

    @pl.when(first)
    def _init():
        o_ref[0, 0] = 0

    o_ref[0, 0] += cnt


def kernel(input, target):
    batch, num_classes, graph = input.shape
    nw = _NC * _NS

    sc_body = functools.partial(
        _sc_body, num_classes=num_classes, graph=graph, n_batches=_BSC)
    sc_partials = pl.kernel(
        sc_body,
        out_type=jax.ShapeDtypeStruct((nw, _L), jnp.int32),
        scratch_types=[
            pltpu.VMEM((num_classes, _CHUNK), jnp.float32),
            pltpu.VMEM((num_classes, _CHUNK), jnp.float32),
            pltpu.VMEM((_CHUNK,), jnp.int32),
            pltpu.VMEM((_CHUNK,), jnp.int32),
            pltpu.VMEM((_L,), jnp.int32),
            pltpu.SemaphoreType.DMA,
            pltpu.SemaphoreType.DMA,
        ],
        mesh=plsc.VectorSubcoreMesh(core_axis_name="c", subcore_axis_name="s"),
    )(input, target)

    t3 = target.reshape(batch, 1, graph)
    n_tc = batch - _BSC
    nj = graph // _W
    tc_cnt = pl.pallas_call(
        functools.partial(_tc_kernel, num_classes=num_classes),
        grid=(n_tc, nj),
        in_specs=[
            pl.BlockSpec((1, num_classes, _W), lambda b, j: (_BSC + b, 0, j)),
            pl.BlockSpec((1, 1, _W), lambda b, j: (_BSC + b, 0, j)),
        ],
        out_specs=pl.BlockSpec(
            (1, 1), lambda b, j: (0, 0), memory_space=pltpu.SMEM
        ),
        out_shape=jax.ShapeDtypeStruct((1, 1), jnp.int32),
    )(input, t3)

    cnt = jnp.sum(sc_partials) + tc_cnt[0, 0]
    edge_acc = cnt.astype(jnp.float32) / float(target.size)
    return 1.0 - edge_acc
